# Initial kernel scaffold; baseline (speedup 1.0000x reference)
import jax
import jax.numpy as jnp
from jax.experimental import pallas as pl


def kernel(x, memory, W_q, b_q):
    raise NotImplementedError("write your pallas kernel here")


# speedup vs baseline: 17.8295x; 17.8295x over previous
import jax
import jax.numpy as jnp
from jax import lax
from jax.experimental import pallas as pl
from jax.experimental.pallas import tpu as pltpu


def _proj_body(x_ref, wq_ref, bq_ref, q_ref):
    q = lax.dot_general(
        x_ref[...], wq_ref[...],
        (((1,), (1,)), ((), ())),
        preferred_element_type=jnp.float32,
    )
    q_ref[...] = q + bq_ref[...]


def _read_body(q_ref, mem_ref, rv_ref, pos_ref):
    mem = mem_ref[0]
    q = q_ref[0]
    m_sz = mem.shape[0]
    r_sz = q.shape[0]
    k_sz = pos_ref.shape[-1]

    qn = q / (jnp.sqrt(jnp.sum(q * q, axis=-1, keepdims=True)) + 1e-8)
    mn = mem / (jnp.sqrt(jnp.sum(mem * mem, axis=-1, keepdims=True)) + 1e-8)

    sim = lax.dot_general(
        qn, mn,
        (((1,), (1,)), ((), ())),
        preferred_element_type=jnp.float32,
    )

    iota_m = lax.broadcasted_iota(jnp.int32, (r_sz, m_sz), 1)
    neg_inf = jnp.float32(-jnp.inf)

    wacc = jnp.zeros((r_sz, m_sz), dtype=jnp.float32)
    denom = jnp.zeros((r_sz, 1), dtype=jnp.float32)
    v0 = None
    cols = []
    for _ in range(k_sz):
        vmax = jnp.max(sim, axis=1, keepdims=True)
        idx = jnp.min(jnp.where(sim == vmax, iota_m, m_sz),
                      axis=1, keepdims=True)
        cols.append(idx)
        if v0 is None:
            v0 = vmax
        w = jnp.exp(vmax - v0)
        onehot = iota_m == idx
        wacc = jnp.where(onehot, w, wacc)
        denom = denom + w
        sim = jnp.where(onehot, neg_inf, sim)

    pos_ref[0] = jnp.concatenate(cols, axis=1)

    rv = lax.dot_general(
        wacc, mem,
        (((1,), (0,)), ((), ())),
        preferred_element_type=jnp.float32,
    )
    rv_ref[0] = rv / denom


def kernel(x, memory, W_q, b_q):
    b, m, w = memory.shape
    wr = W_q.shape[0]
    r = wr // w
    k = 8

    q = pl.pallas_call(
        _proj_body,
        out_shape=jax.ShapeDtypeStruct((b, wr), jnp.float32),
    )(x, W_q, b_q.reshape(1, wr))
    q3 = q.reshape(b, r, w)

    read_vectors, read_positions = pl.pallas_call(
        _read_body,
        grid=(b,),
        in_specs=[
            pl.BlockSpec((1, r, w), lambda i: (i, 0, 0)),
            pl.BlockSpec((1, m, w), lambda i: (i, 0, 0)),
        ],
        out_specs=[
            pl.BlockSpec((1, r, w), lambda i: (i, 0, 0)),
            pl.BlockSpec((1, r, k), lambda i: (i, 0, 0)),
        ],
        out_shape=[
            jax.ShapeDtypeStruct((b, r, w), jnp.float32),
            jax.ShapeDtypeStruct((b, r, k), jnp.int32),
        ],
    )(q3, memory)

    return read_vectors, read_positions


# speedup vs baseline: 17.9495x; 1.0067x over previous
import jax
import jax.numpy as jnp
from jax import lax
from jax.experimental import pallas as pl
from jax.experimental.pallas import tpu as pltpu
from jax.experimental.pallas import tpu_sc as plsc

_K = 8
_NC = 2
_NS = 16
_L = 16


def _proj_body(x_ref, wq_ref, bq_ref, q_ref):
    q = lax.dot_general(
        x_ref[...], wq_ref[...],
        (((1,), (1,)), ((), ())),
        preferred_element_type=jnp.float32,
    )
    q_ref[...] = q + bq_ref[...]


def _topk_body(q_ref, mem_ref, pos_ref, wts_ref):
    mem = mem_ref[0]
    q = q_ref[0]
    m_sz = mem.shape[0]
    r_sz = q.shape[0]

    qn = q / (jnp.sqrt(jnp.sum(q * q, axis=-1, keepdims=True)) + 1e-8)
    mn = mem / (jnp.sqrt(jnp.sum(mem * mem, axis=-1, keepdims=True)) + 1e-8)

    sim = lax.dot_general(
        qn, mn,
        (((1,), (1,)), ((), ())),
        preferred_element_type=jnp.float32,
    )

    iota_m = lax.broadcasted_iota(jnp.int32, (r_sz, m_sz), 1)
    neg_inf = jnp.float32(-jnp.inf)

    denom = jnp.zeros((r_sz, 1), dtype=jnp.float32)
    v0 = None
    cols, ws = [], []
    for _ in range(_K):
        vmax = jnp.max(sim, axis=1, keepdims=True)
        idx = jnp.min(jnp.where(sim == vmax, iota_m, m_sz),
                      axis=1, keepdims=True)
        cols.append(idx)
        if v0 is None:
            v0 = vmax
        w = jnp.exp(vmax - v0)
        ws.append(w)
        denom = denom + w
        sim = jnp.where(iota_m == idx, neg_inf, sim)

    pos_ref[0] = jnp.concatenate(cols, axis=1)
    wts_ref[0] = jnp.concatenate(ws, axis=1) / denom


def _sc_read_body(mem_ref, pos_ref, wts_ref, out_ref,
                  idxv, wvx, rows, acc, sem):
    cid = lax.axis_index("c")
    sid = lax.axis_index("s")
    b = sid * _NC + cid
    m_sz = mem_ref.shape[0] // (_NC * _NS)
    rk = idxv.shape[0]

    pltpu.sync_copy(pos_ref.at[b], idxv)
    pltpu.sync_copy(wts_ref.at[b], wvx)
    for j in range(rk // _L):
        sl = pl.ds(j * _L, _L)
        idxv[sl] = idxv[sl] + b * m_sz
    pltpu.async_copy(mem_ref.at[idxv], rows, sem).wait()

    r_sz, w_sz = acc.shape
    for r in range(r_sz):
        accs = [jnp.zeros((_L,), jnp.float32) for _ in range(w_sz // _L)]
        for k in range(_K):
            wspl = wvx[r * _K + k]
            for c in range(w_sz // _L):
                accs[c] = accs[c] + wspl * rows[r * _K + k, pl.ds(c * _L, _L)]
        for c in range(w_sz // _L):
            acc[r, pl.ds(c * _L, _L)] = accs[c]
    pltpu.sync_copy(acc, out_ref.at[b])


def kernel(x, memory, W_q, b_q):
    b, m, w = memory.shape
    wr = W_q.shape[0]
    r = wr // w

    q = pl.pallas_call(
        _proj_body,
        out_shape=jax.ShapeDtypeStruct((b, wr), jnp.float32),
    )(x, W_q, b_q.reshape(1, wr))
    q3 = q.reshape(b, r, w)

    read_positions, weights = pl.pallas_call(
        _topk_body,
        grid=(b,),
        in_specs=[
            pl.BlockSpec((1, r, w), lambda i: (i, 0, 0)),
            pl.BlockSpec((1, m, w), lambda i: (i, 0, 0)),
        ],
        out_specs=[
            pl.BlockSpec((1, r, _K), lambda i: (i, 0, 0)),
            pl.BlockSpec((1, r, _K), lambda i: (i, 0, 0)),
        ],
        out_shape=[
            jax.ShapeDtypeStruct((b, r, _K), jnp.int32),
            jax.ShapeDtypeStruct((b, r, _K), jnp.float32),
        ],
    )(q3, memory)

    rk = r * _K
    wts_splat = jnp.broadcast_to(weights.reshape(b, rk, 1), (b, rk, _L))
    read_vectors = pl.kernel(
        _sc_read_body,
        out_type=jax.ShapeDtypeStruct((b, r, w), jnp.float32),
        mesh=plsc.VectorSubcoreMesh(core_axis_name="c", subcore_axis_name="s"),
        scratch_types=[
            pltpu.VMEM((rk,), jnp.int32),
            pltpu.VMEM((rk, _L), jnp.float32),
            pltpu.VMEM((rk, w), jnp.float32),
            pltpu.VMEM((r, w), jnp.float32),
            pltpu.SemaphoreType.DMA,
        ],
    )(memory.reshape(b * m, w),
      read_positions.reshape(b, rk),
      wts_splat)

    return read_vectors, read_positions


# speedup vs baseline: 20.6149x; 1.1485x over previous
import jax
import jax.numpy as jnp
from jax import lax
from jax.experimental import pallas as pl
from jax.experimental.pallas import tpu as pltpu
from jax.experimental.pallas import tpu_sc as plsc

_K = 8
_NC = 2
_NS = 16
_L = 16


def _proj_body(x_ref, wq_ref, bq_ref, q_ref):
    q = lax.dot_general(
        x_ref[...], wq_ref[...],
        (((1,), (1,)), ((), ())),
        preferred_element_type=jnp.float32,
    )
    q_ref[...] = q + bq_ref[...]


def _topk_body(q_ref, mem_ref, pos_ref, wts_ref):
    mem = mem_ref[0]
    q = q_ref[0]
    m_sz = mem.shape[0]
    r_sz = q.shape[0]

    w_sz = mem.shape[1]
    qn = q / (jnp.sqrt(jnp.sum(q * q, axis=-1, keepdims=True)) + 1e-8)
    ssq = lax.dot_general(
        jnp.ones((1, w_sz), jnp.float32), mem * mem,
        (((1,), (1,)), ((), ())),
        preferred_element_type=jnp.float32,
    )
    inv = 1.0 / (jnp.sqrt(ssq) + 1e-8)

    sim = lax.dot_general(
        qn, mem,
        (((1,), (1,)), ((), ())),
        preferred_element_type=jnp.float32,
    ) * inv

    iota_m = lax.broadcasted_iota(jnp.int32, (r_sz, m_sz), 1)
    neg_inf = jnp.float32(-jnp.inf)

    denom = jnp.zeros((r_sz, 1), dtype=jnp.float32)
    v0 = None
    cols, ws = [], []
    for _ in range(_K):
        vmax = jnp.max(sim, axis=1, keepdims=True)
        idx = jnp.min(jnp.where(sim == vmax, iota_m, m_sz),
                      axis=1, keepdims=True)
        cols.append(idx)
        if v0 is None:
            v0 = vmax
        w = jnp.exp(vmax - v0)
        ws.append(w)
        denom = denom + w
        sim = jnp.where(iota_m == idx, neg_inf, sim)

    pos_ref[0] = jnp.concatenate(cols, axis=1)
    wts_ref[0] = jnp.concatenate(ws, axis=1) / denom


def _sc_read_body(mem_ref, pos_ref, wts_ref, out_ref,
                  idxv, wvx, rows, acc, sem):
    cid = lax.axis_index("c")
    sid = lax.axis_index("s")
    b = sid * _NC + cid
    m_sz = mem_ref.shape[0] // (_NC * _NS)
    rk = idxv.shape[0]

    pltpu.sync_copy(pos_ref.at[b], idxv)
    pltpu.sync_copy(wts_ref.at[b], wvx)
    for j in range(rk // _L):
        sl = pl.ds(j * _L, _L)
        idxv[sl] = idxv[sl] + b * m_sz
    pltpu.async_copy(mem_ref.at[idxv], rows, sem).wait()

    r_sz, w_sz = acc.shape
    for r in range(r_sz):
        accs = [jnp.zeros((_L,), jnp.float32) for _ in range(w_sz // _L)]
        for k in range(_K):
            wspl = wvx[r * _K + k]
            for c in range(w_sz // _L):
                accs[c] = accs[c] + wspl * rows[r * _K + k, pl.ds(c * _L, _L)]
        for c in range(w_sz // _L):
            acc[r, pl.ds(c * _L, _L)] = accs[c]
    pltpu.sync_copy(acc, out_ref.at[b])


def kernel(x, memory, W_q, b_q):
    b, m, w = memory.shape
    wr = W_q.shape[0]
    r = wr // w

    q = pl.pallas_call(
        _proj_body,
        out_shape=jax.ShapeDtypeStruct((b, wr), jnp.float32),
    )(x, W_q, b_q.reshape(1, wr))
    q3 = q.reshape(b, r, w)

    read_positions, weights = pl.pallas_call(
        _topk_body,
        grid=(b,),
        in_specs=[
            pl.BlockSpec((1, r, w), lambda i: (i, 0, 0)),
            pl.BlockSpec((1, m, w), lambda i: (i, 0, 0)),
        ],
        out_specs=[
            pl.BlockSpec((1, r, _K), lambda i: (i, 0, 0)),
            pl.BlockSpec((1, r, _K), lambda i: (i, 0, 0)),
        ],
        out_shape=[
            jax.ShapeDtypeStruct((b, r, _K), jnp.int32),
            jax.ShapeDtypeStruct((b, r, _K), jnp.float32),
        ],
    )(q3, memory)

    rk = r * _K
    wts_splat = jnp.broadcast_to(weights.reshape(b, rk, 1), (b, rk, _L))
    read_vectors = pl.kernel(
        _sc_read_body,
        out_type=jax.ShapeDtypeStruct((b, r, w), jnp.float32),
        mesh=plsc.VectorSubcoreMesh(core_axis_name="c", subcore_axis_name="s"),
        scratch_types=[
            pltpu.VMEM((rk,), jnp.int32),
            pltpu.VMEM((rk, _L), jnp.float32),
            pltpu.VMEM((rk, w), jnp.float32),
            pltpu.VMEM((r, w), jnp.float32),
            pltpu.SemaphoreType.DMA,
        ],
    )(memory.reshape(b * m, w),
      read_positions.reshape(b, rk),
      wts_splat)

    return read_vectors, read_positions
